# baseline (speedup 1.0000x reference)
import functools

import jax
import jax.numpy as jnp
from jax.experimental import pallas as pl
from jax.experimental.pallas import tpu as pltpu

_N = 10000
_E = 160000
_H1 = 64
_HID = 64
_H2 = 5
_DOUT = 128


def _mm_kernel(a_ref, b_ref, o_ref):
    o_ref[...] = jnp.dot(a_ref[...], b_ref[...],
                         preferred_element_type=jnp.float32)


def _mm(a, b, bm):
    m, k = a.shape
    k2, n = b.shape
    assert k == k2 and m % bm == 0
    return pl.pallas_call(
        _mm_kernel,
        grid=(m // bm,),
        in_specs=[
            pl.BlockSpec((bm, k), lambda i: (i, 0)),
            pl.BlockSpec((k, n), lambda i: (0, 0)),
        ],
        out_specs=pl.BlockSpec((bm, n), lambda i: (i, 0)),
        out_shape=jax.ShapeDtypeStruct((m, n), jnp.float32),
    )(a, b)


def _gat_layer_jnp(h, a_src, a_dst, src, dst, H, C):
    h = h.reshape(-1, H, C)
    alpha_src = jnp.sum(h * a_src, axis=-1)
    alpha_dst = jnp.sum(h * a_dst, axis=-1)
    alpha = alpha_src[src] + alpha_dst[dst]
    alpha = jax.nn.leaky_relu(alpha, negative_slope=0.2)
    amax = jax.ops.segment_max(alpha, dst, num_segments=_N)
    amax = jnp.where(jnp.isfinite(amax), amax, 0.0)
    ex = jnp.exp(alpha - amax[dst])
    denom = jax.ops.segment_sum(ex, dst, num_segments=_N)[dst] + 1e-16
    coeff = ex / denom
    out = jax.ops.segment_sum(coeff[:, :, None] * h[src], dst, num_segments=_N)
    return out


def kernel(x, edge_index, lefts, rights, W1, att_src1, att_dst1, b1,
           W2, att_src2, att_dst2, b2):
    src = edge_index[0]
    dst = edge_index[1]

    h = _mm(x, W1, 400)
    o1 = _gat_layer_jnp(h, att_src1, att_dst1, src, dst, _H1, _HID)
    h1 = jax.nn.elu(o1.reshape(-1, _H1 * _HID) + b1)

    hp = _mm(h1, W2, 400)
    o2 = _gat_layer_jnp(hp, att_src2, att_dst2, src, dst, _H2, _DOUT)
    h2 = o2.mean(axis=1) + b2

    score = jnp.sum(h2[lefts] + h2[rights], axis=-1)
    return jax.nn.softmax(score, axis=0)
